# baseline (speedup 1.0000x reference)
import functools

import jax
import jax.numpy as jnp
from jax import lax
from jax.experimental import pallas as pl
from jax.experimental.pallas import tpu as pltpu
from jax.experimental.pallas import tpu_sc as plsc

NUM_ITEMS = 1000000
EMBED = 46
GENRE = 18
OUT_D = EMBED + GENRE
BATCH = 16384

_NC = 2
_NS = 16
_NW = _NC * _NS
_BPW = BATCH // _NW
_GRP = 8
_NPAIR = _BPW // (2 * _GRP)


def _body(idx_hbm, table_hbm, out_hbm, idx_v, buf_a, buf_b, out_v,
          sem_a, sem_b):
    wid = lax.axis_index("s") * _NC + lax.axis_index("c")
    base = pl.multiple_of(wid * _BPW, _BPW)

    pltpu.sync_copy(idx_hbm.at[pl.ds(base, _BPW)], idx_v)

    def issue(idx16, lane0, buf, sem):
        for k in range(_GRP):
            i = idx16[lane0 + k]
            t = pl.multiple_of((i // 8) * 8, 8)
            pltpu.async_copy(table_hbm.at[pl.ds(t, 8), :], buf.at[k], sem)

    def drain_consume(idx16, lane0, row0, buf, sem):
        for k in range(_GRP):
            pltpu.make_async_copy(table_hbm.at[pl.ds(0, 8), :], buf.at[k],
                                  sem).wait()
        z = jnp.zeros((16,), jnp.float32)
        for k in range(_GRP):
            i = idx16[lane0 + k]
            r = lax.rem(i, 8)
            g = row0 + k
            out_v[g, pl.ds(0, 16)] = buf[k, r, pl.ds(0, 16)]
            out_v[g, pl.ds(16, 16)] = buf[k, r, pl.ds(16, 16)]
            out_v[g, pl.ds(30, 16)] = buf[k, r, pl.ds(30, 16)]
            out_v[g, pl.ds(EMBED, 16)] = z
            out_v[g, pl.ds(OUT_D - 16, 16)] = z

    idx16_0 = idx_v[pl.ds(0, 16)]
    issue(idx16_0, 0, buf_a, sem_a)
    issue(idx16_0, _GRP, buf_b, sem_b)

    def pair(p, carry):
        row0 = p * 16
        idx16_cur = idx_v[pl.ds(row0, 16)]
        idx16_nxt = idx_v[pl.ds(row0 + 16, 16)]
        drain_consume(idx16_cur, 0, row0, buf_a, sem_a)
        issue(idx16_nxt, 0, buf_a, sem_a)
        drain_consume(idx16_cur, _GRP, row0 + _GRP, buf_b, sem_b)
        issue(idx16_nxt, _GRP, buf_b, sem_b)
        return carry

    lax.fori_loop(0, _NPAIR - 1, pair, 0)

    row0 = (_NPAIR - 1) * 16
    idx16_l = idx_v[pl.ds(row0, 16)]
    drain_consume(idx16_l, 0, row0, buf_a, sem_a)
    drain_consume(idx16_l, _GRP, row0 + _GRP, buf_b, sem_b)

    pltpu.sync_copy(out_v, out_hbm.at[pl.ds(base, _BPW), :])


@functools.partial(jax.jit)
def kernel(item_inputs, table):
    idx = item_inputs.astype(jnp.int32)
    run = pl.kernel(
        _body,
        out_type=jax.ShapeDtypeStruct((BATCH, OUT_D), jnp.float32),
        mesh=plsc.VectorSubcoreMesh(core_axis_name="c", subcore_axis_name="s"),
        scratch_types=[
            pltpu.VMEM((_BPW,), jnp.int32),
            pltpu.VMEM((_GRP, 8, EMBED), jnp.float32),
            pltpu.VMEM((_GRP, 8, EMBED), jnp.float32),
            pltpu.VMEM((_BPW, OUT_D), jnp.float32),
            pltpu.SemaphoreType.DMA,
            pltpu.SemaphoreType.DMA,
        ],
        compiler_params=pltpu.CompilerParams(use_tc_tiling_on_sc=True),
    )
    return run(idx, table)
